# baseline (speedup 1.0000x reference)
import jax
import jax.numpy as jnp
from jax import lax
from jax.experimental import pallas as pl
from jax.experimental.pallas import tpu as pltpu
from jax.experimental.pallas import tpu_sc as plsc

B, D, H, W = 16, 128, 128, 128
C = D // 2
HW = H * W
P = B * C
L = 16
G = 128
NG = 2 * P // G


def _sc_body(out_hbm, ind_hbm, t_hbm, mask_hbm, loss_hbm,
             ind_v, idx_v, vals_v, t_v, mask_v, out_v, sem):
    cid = lax.axis_index("c")
    sid = lax.axis_index("s")

    @pl.when((cid == 0) & (sid == 0))
    def _work():
        pltpu.sync_copy(ind_hbm, ind_v)
        pltpu.sync_copy(t_hbm, t_v)
        pltpu.sync_copy(mask_hbm, mask_v)

        iota = lax.iota(jnp.int32, L)
        for j in range(P // L):
            pair = j * L + iota
            f0 = pair * (2 * HW) + ind_v[pl.ds(j * L, L)]
            idx_v[pl.ds(j * L, L)] = f0
            idx_v[pl.ds(P + j * L, L)] = f0 + HW

        copies = []
        for g in range(NG):
            copies.append(pltpu.make_async_copy(
                out_hbm.at[idx_v.at[pl.ds(g * G, G)]],
                vals_v.at[pl.ds(g * G, G)],
                sem,
            ))
        for cp in copies:
            cp.start()
        for cp in copies:
            cp.wait()

        acc = jnp.zeros((L,), jnp.float32)
        for j in range(2 * P // L):
            v = vals_v[pl.ds(j * L, L)]
            t = t_v[pl.ds(j * L, L)]
            m = mask_v[pl.ds((j % (P // L)) * L, L)]
            acc = acc + jnp.abs(v * m - t * m)
        macc = jnp.zeros((L,), jnp.float32)
        for j in range(P // L):
            macc = macc + mask_v[pl.ds(j * L, L)]

        num = jnp.float32(0.0)
        den = jnp.float32(0.0)
        for k in range(L):
            num = num + acc[k]
            den = den + macc[k]
        nv = jnp.full((L,), num, jnp.float32)
        dv = jnp.full((L,), 2.0 * den + 0.0001, jnp.float32)
        out_v[...] = nv / dv
        pltpu.sync_copy(out_v, loss_hbm)


@jax.jit
def _sc_loss(out_flat, ind_flat, t_cat, mask_flat):
    mesh = plsc.VectorSubcoreMesh(core_axis_name="c", subcore_axis_name="s")
    loss = pl.kernel(
        _sc_body,
        out_type=jax.ShapeDtypeStruct((L,), jnp.float32),
        mesh=mesh,
        scratch_types=[
            pltpu.VMEM((P,), jnp.int32),
            pltpu.VMEM((2 * P,), jnp.int32),
            pltpu.VMEM((2 * P,), jnp.float32),
            pltpu.VMEM((2 * P,), jnp.float32),
            pltpu.VMEM((P,), jnp.float32),
            pltpu.VMEM((L,), jnp.float32),
            pltpu.SemaphoreType.DMA,
        ],
    )(out_flat, ind_flat, t_cat, mask_flat)
    return loss[0]


def kernel(output, mask, ind, target):
    assert output.shape == (B, D, H, W)
    out_flat = output.reshape(B * D * H * W)
    ind_flat = ind.reshape(P).astype(jnp.int32)
    mask_flat = mask.reshape(P).astype(jnp.float32)
    t = target.astype(jnp.float32)
    t_cat = jnp.concatenate([t[:, :, 0].reshape(P), t[:, :, 1].reshape(P)])
    return _sc_loss(out_flat, ind_flat, t_cat, mask_flat)


# speedup vs baseline: 1.0689x; 1.0689x over previous
import jax
import jax.numpy as jnp
from jax import lax
from jax.experimental import pallas as pl
from jax.experimental.pallas import tpu as pltpu
from jax.experimental.pallas import tpu_sc as plsc

B, D, H, W = 16, 128, 128, 128
C = D // 2
HW = H * W
P = B * C
L = 16
G = 128
NG = 2 * P // G


def _sc_body(out_hbm, ind_hbm, t_hbm, mask_hbm, loss_hbm,
             ind_v, idx_v, vals_v, t_v, mask_v, out_v, sem):
    cid = lax.axis_index("c")
    sid = lax.axis_index("s")

    @pl.when((cid == 0) & (sid == 0))
    def _work():
        pltpu.sync_copy(ind_hbm, ind_v)
        pltpu.sync_copy(t_hbm, t_v)
        pltpu.sync_copy(mask_hbm, mask_v)

        iota = lax.iota(jnp.int32, L)
        for j in range(P // L):
            pair = j * L + iota
            f0 = pair * (2 * HW) + ind_v[pl.ds(j * L, L)]
            idx_v[pl.ds(j * L, L)] = f0
            idx_v[pl.ds(P + j * L, L)] = f0 + HW

        copies = []
        for g in range(NG):
            copies.append(pltpu.make_async_copy(
                out_hbm.at[idx_v.at[pl.ds(g * G, G)]],
                vals_v.at[pl.ds(g * G, G)],
                sem,
            ))
        for cp in copies:
            cp.start()
        for cp in copies:
            cp.wait()

        acc = jnp.zeros((L,), jnp.float32)
        for j in range(2 * P // L):
            v = vals_v[pl.ds(j * L, L)]
            t = t_v[pl.ds(j * L, L)]
            m = mask_v[pl.ds((j % (P // L)) * L, L)]
            acc = acc + jnp.abs(v * m - t * m)
        macc = jnp.zeros((L,), jnp.float32)
        for j in range(P // L):
            macc = macc + mask_v[pl.ds(j * L, L)]

        num = jnp.float32(0.0)
        den = jnp.float32(0.0)
        for k in range(L):
            num = num + acc[k]
            den = den + macc[k]
        nv = jnp.full((L,), num, jnp.float32)
        dv = jnp.full((L,), 2.0 * den + 0.0001, jnp.float32)
        out_v[...] = nv / dv
        pltpu.sync_copy(out_v, loss_hbm)


@jax.jit
def _sc_loss(out_flat, ind_flat, t_cat, mask_flat):
    mesh = plsc.VectorSubcoreMesh(
        core_axis_name="c", subcore_axis_name="s", num_cores=1)
    loss = pl.kernel(
        _sc_body,
        out_type=jax.ShapeDtypeStruct((L,), jnp.float32),
        mesh=mesh,
        scratch_types=[
            pltpu.VMEM((P,), jnp.int32),
            pltpu.VMEM((2 * P,), jnp.int32),
            pltpu.VMEM((2 * P,), jnp.float32),
            pltpu.VMEM((2 * P,), jnp.float32),
            pltpu.VMEM((P,), jnp.float32),
            pltpu.VMEM((L,), jnp.float32),
            pltpu.SemaphoreType.DMA,
        ],
    )(out_flat, ind_flat, t_cat, mask_flat)
    return loss[0]


def kernel(output, mask, ind, target):
    assert output.shape == (B, D, H, W)
    out_flat = output.reshape(B * D * H * W)
    ind_flat = ind.reshape(P).astype(jnp.int32)
    mask_flat = mask.reshape(P).astype(jnp.float32)
    t = target.astype(jnp.float32)
    t_cat = jnp.concatenate([t[:, :, 0].reshape(P), t[:, :, 1].reshape(P)])
    return _sc_loss(out_flat, ind_flat, t_cat, mask_flat)


# speedup vs baseline: 1.2054x; 1.1276x over previous
import jax
import jax.numpy as jnp
from jax import lax
from jax.experimental import pallas as pl
from jax.experimental.pallas import tpu as pltpu
from jax.experimental.pallas import tpu_sc as plsc

B, D, H, W = 16, 128, 128, 128
C = D // 2
HW = H * W
P = B * C
NW = 16
PPW = P // NW
L = 16


def _sc_body(out_hbm, ind_hbm, t_hbm, mask_hbm, part_hbm,
             ind_v, idx_v, vals_v, t_v, mask_v, acc_v,
             sem_i, sem_t, sem_g):
    sid = lax.axis_index("s")
    base = sid * PPW

    cp_ind = pltpu.make_async_copy(ind_hbm.at[pl.ds(base, PPW)], ind_v, sem_i)
    cp_t0 = pltpu.make_async_copy(
        t_hbm.at[pl.ds(base, PPW)], t_v.at[pl.ds(0, PPW)], sem_t)
    cp_t1 = pltpu.make_async_copy(
        t_hbm.at[pl.ds(P + base, PPW)], t_v.at[pl.ds(PPW, PPW)], sem_t)
    cp_m = pltpu.make_async_copy(
        mask_hbm.at[pl.ds(base, PPW)], mask_v, sem_t)
    cp_ind.start()
    cp_t0.start()
    cp_t1.start()
    cp_m.start()

    cp_ind.wait()
    iota = lax.iota(jnp.int32, L)
    for j in range(PPW // L):
        pair = base + j * L + iota
        f0 = pair * (2 * HW) + ind_v[pl.ds(j * L, L)]
        idx_v[pl.ds(j * L, L)] = f0
        idx_v[pl.ds(PPW + j * L, L)] = f0 + HW

    cp_g = pltpu.make_async_copy(out_hbm.at[idx_v], vals_v, sem_g)
    cp_g.start()
    cp_t0.wait()
    cp_t1.wait()
    cp_m.wait()
    cp_g.wait()

    acc = jnp.zeros((L,), jnp.float32)
    for j in range(2 * PPW // L):
        v = vals_v[pl.ds(j * L, L)]
        t = t_v[pl.ds(j * L, L)]
        m = mask_v[pl.ds((j % (PPW // L)) * L, L)]
        acc = acc + jnp.abs(v * m - t * m)
    macc = jnp.zeros((L,), jnp.float32)
    for j in range(PPW // L):
        macc = macc + mask_v[pl.ds(j * L, L)]
    acc_v[pl.ds(0, L)] = acc
    acc_v[pl.ds(L, L)] = macc
    pltpu.sync_copy(acc_v, part_hbm.at[sid])


def _tc_reduce(part_ref, out_ref):
    p = part_ref[...]
    num = jnp.sum(p[:, :L])
    den = jnp.sum(p[:, L:])
    out_ref[...] = jnp.full((1, 1), num / (2.0 * den + 0.0001), jnp.float32)


@jax.jit
def _sc_loss(out_flat, ind_flat, t_cat, mask_flat):
    mesh = plsc.VectorSubcoreMesh(
        core_axis_name="c", subcore_axis_name="s", num_cores=1)
    part = pl.kernel(
        _sc_body,
        out_type=jax.ShapeDtypeStruct((NW, 2 * L), jnp.float32),
        mesh=mesh,
        scratch_types=[
            pltpu.VMEM((PPW,), jnp.int32),
            pltpu.VMEM((2 * PPW,), jnp.int32),
            pltpu.VMEM((2 * PPW,), jnp.float32),
            pltpu.VMEM((2 * PPW,), jnp.float32),
            pltpu.VMEM((PPW,), jnp.float32),
            pltpu.VMEM((2 * L,), jnp.float32),
            pltpu.SemaphoreType.DMA,
            pltpu.SemaphoreType.DMA,
            pltpu.SemaphoreType.DMA,
        ],
    )(out_flat, ind_flat, t_cat, mask_flat)
    loss = pl.pallas_call(
        _tc_reduce,
        out_shape=jax.ShapeDtypeStruct((1, 1), jnp.float32),
    )(part)
    return loss[0, 0]


def kernel(output, mask, ind, target):
    assert output.shape == (B, D, H, W)
    out_flat = output.reshape(B * D * H * W)
    ind_flat = ind.reshape(P).astype(jnp.int32)
    mask_flat = mask.reshape(P).astype(jnp.float32)
    t = target.astype(jnp.float32)
    t_cat = jnp.concatenate([t[:, :, 0].reshape(P), t[:, :, 1].reshape(P)])
    return _sc_loss(out_flat, ind_flat, t_cat, mask_flat)


# speedup vs baseline: 1.2862x; 1.0671x over previous
import jax
import jax.numpy as jnp
from jax import lax
from jax.experimental import pallas as pl
from jax.experimental.pallas import tpu as pltpu
from jax.experimental.pallas import tpu_sc as plsc

B, D, H, W = 16, 128, 128, 128
C = D // 2
HW = H * W
P = B * C
NW = 16
PPW = P // NW
L = 16
ROW = 4 * PPW


def _sc_body(out_hbm, staged_hbm, part_hbm,
             st_v, idx_v, vals_v, acc_v, sem_s, sem_g):
    sid = lax.axis_index("s")

    cp_st = pltpu.make_async_copy(staged_hbm.at[sid], st_v, sem_s)
    cp_st.start()
    cp_st.wait()

    base = sid * PPW
    iota = lax.iota(jnp.int32, L)
    for j in range(PPW // L):
        pair = base + j * L + iota
        ind = st_v[pl.ds(j * L, L)].astype(jnp.int32)
        f0 = pair * (2 * HW) + ind
        idx_v[pl.ds(j * L, L)] = f0
        idx_v[pl.ds(PPW + j * L, L)] = f0 + HW

    cp_g = pltpu.make_async_copy(out_hbm.at[idx_v], vals_v, sem_g)
    cp_g.start()
    cp_g.wait()

    acc = jnp.zeros((L,), jnp.float32)
    for j in range(2 * PPW // L):
        v = vals_v[pl.ds(j * L, L)]
        t = st_v[pl.ds(PPW + j * L, L)]
        m = st_v[pl.ds(3 * PPW + (j % (PPW // L)) * L, L)]
        acc = acc + jnp.abs(v * m - t * m)
    macc = jnp.zeros((L,), jnp.float32)
    for j in range(PPW // L):
        macc = macc + st_v[pl.ds(3 * PPW + j * L, L)]
    acc_v[pl.ds(0, L)] = acc
    acc_v[pl.ds(L, L)] = macc
    pltpu.sync_copy(acc_v, part_hbm.at[sid])


def _tc_reduce(part_ref, out_ref):
    p = part_ref[...]
    num = jnp.sum(p[:, :L])
    den = jnp.sum(p[:, L:])
    out_ref[...] = jnp.full((1, 1), num / (2.0 * den + 0.0001), jnp.float32)


@jax.jit
def _sc_loss(out_flat, staged):
    mesh = plsc.VectorSubcoreMesh(
        core_axis_name="c", subcore_axis_name="s", num_cores=1)
    part = pl.kernel(
        _sc_body,
        out_type=jax.ShapeDtypeStruct((NW, 2 * L), jnp.float32),
        mesh=mesh,
        scratch_types=[
            pltpu.VMEM((ROW,), jnp.float32),
            pltpu.VMEM((2 * PPW,), jnp.int32),
            pltpu.VMEM((2 * PPW,), jnp.float32),
            pltpu.VMEM((2 * L,), jnp.float32),
            pltpu.SemaphoreType.DMA,
            pltpu.SemaphoreType.DMA,
        ],
    )(out_flat, staged)
    loss = pl.pallas_call(
        _tc_reduce,
        out_shape=jax.ShapeDtypeStruct((1, 1), jnp.float32),
    )(part)
    return loss[0, 0]


def kernel(output, mask, ind, target):
    assert output.shape == (B, D, H, W)
    out_flat = output.reshape(B * D * H * W)
    ind_f = ind.reshape(NW, PPW).astype(jnp.float32)
    t = target.astype(jnp.float32)
    staged = jnp.concatenate(
        [ind_f,
         t[:, :, 0].reshape(NW, PPW),
         t[:, :, 1].reshape(NW, PPW),
         mask.astype(jnp.float32).reshape(NW, PPW)],
        axis=1)
    return _sc_loss(out_flat, staged)
